# baseline (speedup 1.0000x reference)
import functools
import math

import jax
import jax.numpy as jnp
from jax import lax
from jax.experimental import pallas as pl
from jax.experimental.pallas import tpu as pltpu
from jax.experimental.pallas import tpu_sc as plsc

NUM_EMB = 1000000
D = 64
B = 4096
L = 200
NROWS = B * L
NELEM = NROWS * D
ALPHA = 5.0
SCALE = ALPHA / math.sqrt(L * D)

KS0 = 0
KS1 = 12345
KS2 = KS0 ^ KS1 ^ 0x1BD11BDA

_ROT = ((13, 15, 26, 6), (17, 29, 16, 24))


def _rotl(x, r):
    return lax.shift_left(x, jnp.int32(r)) | lax.shift_right_logical(
        x, jnp.int32(32 - r))


def _threefry_bits(cnt):
    ks = (jnp.int32(KS0), jnp.int32(KS1), jnp.int32(KS2))
    x0 = jnp.zeros_like(cnt) + ks[0]
    x1 = cnt + ks[1]
    for i in range(5):
        for r in _ROT[i % 2]:
            x0 = x0 + x1
            x1 = _rotl(x1, r)
            x1 = x1 ^ x0
        x0 = x0 + ks[(i + 1) % 3]
        x1 = x1 + ks[(i + 2) % 3] + jnp.int32(i + 1)
    return x0 ^ x1


_TC_ROWS = 4096


def _noise_add_body(emb_ref, out_ref):
    g = pl.program_id(0)
    r = jax.lax.broadcasted_iota(jnp.int32, (_TC_ROWS, D), 0)
    c = jax.lax.broadcasted_iota(jnp.int32, (_TC_ROWS, D), 1)
    cnt = g * (_TC_ROWS * D) + r * D + c
    bits = _threefry_bits(cnt)
    u = lax.shift_right_logical(bits, jnp.int32(9)) | jnp.int32(0x3F800000)
    eps = lax.bitcast_convert_type(u, jnp.float32) - 1.0
    out_ref[...] = emb_ref[...] + jnp.float32(SCALE) * eps


def _noise_add(emb):
    grid = NROWS // _TC_ROWS
    return pl.pallas_call(
        _noise_add_body,
        grid=(grid,),
        in_specs=[pl.BlockSpec((_TC_ROWS, D), lambda i: (i, 0))],
        out_specs=pl.BlockSpec((_TC_ROWS, D), lambda i: (i, 0)),
        out_shape=jax.ShapeDtypeStruct((NROWS, D), jnp.float32),
    )(emb)


_NW = 32
_ROWS_PER_W = NROWS // _NW
_CHUNK = 1024
_NCHUNK = _ROWS_PER_W // _CHUNK


def _gather_body(idx_hbm, table_hbm, out_hbm, idx_v, rows_v, sem):
    wid = lax.axis_index("s") * 2 + lax.axis_index("c")

    def body(i, carry):
        base = wid * _ROWS_PER_W + i * _CHUNK
        pltpu.sync_copy(idx_hbm.at[pl.ds(base, _CHUNK)], idx_v)
        pltpu.async_copy(table_hbm.at[idx_v], rows_v, sem).wait()
        pltpu.sync_copy(rows_v, out_hbm.at[pl.ds(base, _CHUNK)])
        return carry

    lax.fori_loop(0, _NCHUNK, body, 0)


@functools.cache
def _sc_gather():
    return functools.partial(
        pl.kernel,
        mesh=plsc.VectorSubcoreMesh(core_axis_name="c", subcore_axis_name="s"),
        compiler_params=pltpu.CompilerParams(use_tc_tiling_on_sc=False),
        out_type=jax.ShapeDtypeStruct((NROWS, D), jnp.float32),
        scratch_types=[
            pltpu.VMEM((_CHUNK,), jnp.int32),
            pltpu.VMEM((_CHUNK, D), jnp.float32),
            pltpu.SemaphoreType.DMA,
        ],
    )(_gather_body)


def kernel(xs, table):
    idx = xs.reshape(NROWS).astype(jnp.int32)
    emb = _sc_gather()(idx, table)
    out = _noise_add(emb)
    return out.reshape(B, L, D)


# speedup vs baseline: 1.6669x; 1.6669x over previous
import functools
import math

import jax
import jax.numpy as jnp
from jax import lax
from jax.experimental import pallas as pl
from jax.experimental.pallas import tpu as pltpu
from jax.experimental.pallas import tpu_sc as plsc

NUM_EMB = 1000000
D = 64
B = 4096
L = 200
NROWS = B * L
NELEM = NROWS * D
ALPHA = 5.0
SCALE = ALPHA / math.sqrt(L * D)

KS0 = 0
KS1 = 12345
KS2 = KS0 ^ KS1 ^ 0x1BD11BDA

_ROT = ((13, 15, 26, 6), (17, 29, 16, 24))


def _rotl(x, r):
    return lax.shift_left(x, jnp.int32(r)) | lax.shift_right_logical(
        x, jnp.int32(32 - r))


def _threefry_bits(cnt):
    ks = (jnp.int32(KS0), jnp.int32(KS1), jnp.int32(KS2))
    x0 = jnp.zeros_like(cnt) + ks[0]
    x1 = cnt + ks[1]
    for i in range(5):
        for r in _ROT[i % 2]:
            x0 = x0 + x1
            x1 = _rotl(x1, r)
            x1 = x1 ^ x0
        x0 = x0 + ks[(i + 1) % 3]
        x1 = x1 + ks[(i + 2) % 3] + jnp.int32(i + 1)
    return x0 ^ x1


_TCB = 128
_TCR = _TCB * L * D // 128
_LD = L * D


def _noise_add_body(emb_ref, out_ref):
    g = pl.program_id(0)
    q = jax.lax.broadcasted_iota(jnp.int32, (_TCR, 128), 0)
    j = jax.lax.broadcasted_iota(jnp.int32, (_TCR, 128), 1)
    cnt = g * (_TCR * 128) + q * 128 + j
    bits = _threefry_bits(cnt)
    u = lax.shift_right_logical(bits, jnp.int32(9)) | jnp.int32(0x3F800000)
    eps = lax.bitcast_convert_type(u, jnp.float32) - 1.0
    res = emb_ref[...] + jnp.float32(SCALE) * eps
    r3 = res.reshape(_TCB, _LD // 128, 128)
    out_ref[...] = jnp.transpose(r3, (1, 2, 0)).reshape(_LD, _TCB)


def _noise_add(emb128):
    grid = B // _TCB
    return pl.pallas_call(
        _noise_add_body,
        grid=(grid,),
        in_specs=[pl.BlockSpec((_TCR, 128), lambda i: (i, 0))],
        out_specs=pl.BlockSpec((_LD, _TCB), lambda i: (0, i)),
        out_shape=jax.ShapeDtypeStruct((_LD, B), jnp.float32),
    )(emb128)


_NW = 32
_ROWS_PER_W = NROWS // _NW
_CHUNK = 512
_NCHUNK = _ROWS_PER_W // _CHUNK


def _gather_body(idx_hbm, table_hbm, out_hbm, idx_v, rows_v, gsem):
    wid = lax.axis_index("s") * 2 + lax.axis_index("c")
    base0 = wid * _ROWS_PER_W

    def chunk_slice(i):
        return pl.ds(base0 + i * _CHUNK, _CHUNK)

    pltpu.sync_copy(idx_hbm.at[chunk_slice(0)], idx_v.at[0])
    pltpu.async_copy(table_hbm.at[idx_v.at[0]], rows_v.at[0], gsem)

    def pair(p, carry):
        for b in range(2):
            i = p * 2 + b
            nxt = (b + 1) % 2
            i_nxt = lax.rem(i + 1, _NCHUNK)
            pltpu.sync_copy(idx_hbm.at[pl.ds(base0 + i_nxt * _CHUNK, _CHUNK)],
                            idx_v.at[nxt])
            pltpu.make_async_copy(table_hbm.at[idx_v.at[b]], rows_v.at[b],
                                  gsem).wait()
            pltpu.async_copy(table_hbm.at[idx_v.at[nxt]], rows_v.at[nxt], gsem)
            pltpu.sync_copy(rows_v.at[b], out_hbm.at[chunk_slice(i)])
        return carry

    lax.fori_loop(0, _NCHUNK // 2, pair, 0)
    pltpu.make_async_copy(table_hbm.at[idx_v.at[0]], rows_v.at[0], gsem).wait()


@functools.cache
def _sc_gather():
    return functools.partial(
        pl.kernel,
        mesh=plsc.VectorSubcoreMesh(core_axis_name="c", subcore_axis_name="s"),
        compiler_params=pltpu.CompilerParams(use_tc_tiling_on_sc=False),
        out_type=jax.ShapeDtypeStruct((NROWS, D), jnp.float32),
        scratch_types=[
            pltpu.VMEM((2, _CHUNK), jnp.int32),
            pltpu.VMEM((2, _CHUNK, D), jnp.float32),
            pltpu.SemaphoreType.DMA,
        ],
    )(_gather_body)


def kernel(xs, table):
    idx = xs.reshape(NROWS).astype(jnp.int32)
    emb = _sc_gather()(idx, table)
    out_t = _noise_add(emb.reshape(NROWS // 2, 128))
    return out_t.reshape(L, D, B).transpose(2, 0, 1)


# speedup vs baseline: 1.7746x; 1.0646x over previous
import functools
import math

import jax
import jax.numpy as jnp
from jax import lax
from jax.experimental import pallas as pl
from jax.experimental.pallas import tpu as pltpu
from jax.experimental.pallas import tpu_sc as plsc

NUM_EMB = 1000000
D = 64
B = 4096
L = 200
NROWS = B * L
ALPHA = 5.0
SCALE = ALPHA / math.sqrt(L * D)

KS0 = 0
KS1 = 12345
KS2 = KS0 ^ KS1 ^ 0x1BD11BDA

_ROT = ((13, 15, 26, 6), (17, 29, 16, 24))


def _rotl(x, r):
    return lax.shift_left(x, jnp.int32(r)) | lax.shift_right_logical(
        x, jnp.int32(32 - r))


def _threefry_bits(cnt):
    ks = (jnp.int32(KS0), jnp.int32(KS1), jnp.int32(KS2))
    x0 = jnp.zeros_like(cnt) + ks[0]
    x1 = cnt + ks[1]
    for i in range(5):
        for r in _ROT[i % 2]:
            x0 = x0 + x1
            x1 = _rotl(x1, r)
            x1 = x1 ^ x0
        x0 = x0 + ks[(i + 1) % 3]
        x1 = x1 + ks[(i + 2) % 3] + jnp.int32(i + 1)
    return x0 ^ x1


_TCB = 128
_LD = L * D


def _noise_body(out_ref):
    g = pl.program_id(0)
    r = jax.lax.broadcasted_iota(jnp.int32, (_LD, _TCB), 0)
    c = jax.lax.broadcasted_iota(jnp.int32, (_LD, _TCB), 1)
    cnt = (g * _TCB) * _LD + c * _LD + r
    bits = _threefry_bits(cnt)
    u = lax.shift_right_logical(bits, jnp.int32(9)) | jnp.int32(0x3F800000)
    eps = lax.bitcast_convert_type(u, jnp.float32) - 1.0
    out_ref[...] = jnp.float32(SCALE) * eps


def _noise_t():
    return pl.pallas_call(
        _noise_body,
        grid=(B // _TCB,),
        out_specs=pl.BlockSpec((_LD, _TCB), lambda i: (0, i)),
        out_shape=jax.ShapeDtypeStruct((_LD, B), jnp.float32),
    )()


_TCR = _TCB * _LD // 128


def _add_body(emb_ref, noise_ref, out_ref):
    x = emb_ref[...]
    x3 = x.reshape(_TCB, _LD // 128, 128)
    xt = jnp.transpose(x3, (1, 2, 0)).reshape(_LD, _TCB)
    out_ref[...] = xt + noise_ref[...]


def _add_t(emb128, noise_t):
    return pl.pallas_call(
        _add_body,
        grid=(B // _TCB,),
        in_specs=[pl.BlockSpec((_TCR, 128), lambda i: (i, 0)),
                  pl.BlockSpec((_LD, _TCB), lambda i: (0, i))],
        out_specs=pl.BlockSpec((_LD, _TCB), lambda i: (0, i)),
        out_shape=jax.ShapeDtypeStruct((_LD, B), jnp.float32),
    )(emb128, noise_t)


_NW = 32
_ROWS_PER_W = NROWS // _NW
_CHUNK = 512
_NCHUNK = _ROWS_PER_W // _CHUNK


def _gather_body(idx_hbm, table_hbm, out_hbm, idx_v, rows_v, gsem):
    wid = lax.axis_index("s") * 2 + lax.axis_index("c")
    base0 = wid * _ROWS_PER_W

    def chunk_slice(i):
        return pl.ds(base0 + i * _CHUNK, _CHUNK)

    pltpu.sync_copy(idx_hbm.at[chunk_slice(0)], idx_v.at[0])
    pltpu.async_copy(table_hbm.at[idx_v.at[0]], rows_v.at[0], gsem)

    def pair(p, carry):
        for b in range(2):
            i = p * 2 + b
            nxt = (b + 1) % 2
            i_nxt = lax.rem(i + 1, _NCHUNK)
            pltpu.sync_copy(idx_hbm.at[pl.ds(base0 + i_nxt * _CHUNK, _CHUNK)],
                            idx_v.at[nxt])
            pltpu.make_async_copy(table_hbm.at[idx_v.at[b]], rows_v.at[b],
                                  gsem).wait()
            pltpu.async_copy(table_hbm.at[idx_v.at[nxt]], rows_v.at[nxt], gsem)
            pltpu.sync_copy(rows_v.at[b], out_hbm.at[chunk_slice(i)])
        return carry

    lax.fori_loop(0, _NCHUNK // 2, pair, 0)
    pltpu.make_async_copy(table_hbm.at[idx_v.at[0]], rows_v.at[0], gsem).wait()


@functools.cache
def _sc_gather():
    return functools.partial(
        pl.kernel,
        mesh=plsc.VectorSubcoreMesh(core_axis_name="c", subcore_axis_name="s"),
        compiler_params=pltpu.CompilerParams(use_tc_tiling_on_sc=False),
        out_type=jax.ShapeDtypeStruct((NROWS, D), jnp.float32),
        scratch_types=[
            pltpu.VMEM((2, _CHUNK), jnp.int32),
            pltpu.VMEM((2, _CHUNK, D), jnp.float32),
            pltpu.SemaphoreType.DMA,
        ],
    )(_gather_body)


def kernel(xs, table):
    idx = xs.reshape(NROWS).astype(jnp.int32)
    noise_t = _noise_t()
    emb = _sc_gather()(idx, table)
    out_t = _add_t(emb.reshape(NROWS // 2, 128), noise_t)
    return out_t.reshape(L, D, B).transpose(2, 0, 1)


# speedup vs baseline: 1.7761x; 1.0008x over previous
import functools
import math

import jax
import jax.numpy as jnp
from jax import lax
from jax.experimental import pallas as pl
from jax.experimental.pallas import tpu as pltpu
from jax.experimental.pallas import tpu_sc as plsc

NUM_EMB = 1000000
D = 64
B = 4096
L = 200
NROWS = B * L
ALPHA = 5.0
SCALE = ALPHA / math.sqrt(L * D)

KS0 = 0
KS1 = 12345
KS2 = KS0 ^ KS1 ^ 0x1BD11BDA

_ROT = ((13, 15, 26, 6), (17, 29, 16, 24))


def _rotl(x, r):
    return lax.shift_left(x, jnp.int32(r)) | lax.shift_right_logical(
        x, jnp.int32(32 - r))


def _threefry_bits(cnt):
    ks = (jnp.int32(KS0), jnp.int32(KS1), jnp.int32(KS2))
    x0 = jnp.zeros_like(cnt) + ks[0]
    x1 = cnt + ks[1]
    for i in range(5):
        for r in _ROT[i % 2]:
            x0 = x0 + x1
            x1 = _rotl(x1, r)
            x1 = x1 ^ x0
        x0 = x0 + ks[(i + 1) % 3]
        x1 = x1 + ks[(i + 2) % 3] + jnp.int32(i + 1)
    return x0 ^ x1


_TCB = 128
_LD = L * D


_LDH = _LD // 2


def _noise_half_body(r0, out_ref):
    g = pl.program_id(0)
    r = jax.lax.broadcasted_iota(jnp.int32, (_LDH, _TCB), 0)
    c = jax.lax.broadcasted_iota(jnp.int32, (_LDH, _TCB), 1)
    cnt = (g * _TCB) * _LD + c * _LD + r + r0
    bits = _threefry_bits(cnt)
    u = lax.shift_right_logical(bits, jnp.int32(9)) | jnp.int32(0x3F800000)
    eps = lax.bitcast_convert_type(u, jnp.float32) - 1.0
    out_ref[...] = jnp.float32(SCALE) * eps


def _noise_half(r0):
    return pl.pallas_call(
        functools.partial(_noise_half_body, r0),
        grid=(B // _TCB,),
        out_specs=pl.BlockSpec((_LDH, _TCB), lambda i: (0, i)),
        out_shape=jax.ShapeDtypeStruct((_LDH, B), jnp.float32),
    )()


_TCR = _TCB * _LD // 128


def _add_body(emb_ref, ntop_ref, nbot_ref, out_ref):
    x = emb_ref[...]
    x3 = x.reshape(_TCB, _LD // 128, 128)
    xt = jnp.transpose(x3, (1, 2, 0)).reshape(_LD, _TCB)
    out_ref[pl.ds(0, _LDH), :] = xt[:_LDH, :] + ntop_ref[...]
    out_ref[pl.ds(_LDH, _LDH), :] = xt[_LDH:, :] + nbot_ref[...]


def _add_t(emb128, ntop, nbot):
    return pl.pallas_call(
        _add_body,
        grid=(B // _TCB,),
        in_specs=[pl.BlockSpec((_TCR, 128), lambda i: (i, 0)),
                  pl.BlockSpec((_LDH, _TCB), lambda i: (0, i)),
                  pl.BlockSpec((_LDH, _TCB), lambda i: (0, i))],
        out_specs=pl.BlockSpec((_LD, _TCB), lambda i: (0, i)),
        out_shape=jax.ShapeDtypeStruct((_LD, B), jnp.float32),
    )(emb128, ntop, nbot)


_NW = 32
_ROWS_PER_W = NROWS // _NW
_CHUNK = 512
_NCHUNK = _ROWS_PER_W // _CHUNK


def _gather_body(idx_hbm, table_hbm, out_hbm, idx_v, rows_v, gsem):
    wid = lax.axis_index("s") * 2 + lax.axis_index("c")
    base0 = wid * _ROWS_PER_W

    def chunk_slice(i):
        return pl.ds(base0 + i * _CHUNK, _CHUNK)

    pltpu.sync_copy(idx_hbm.at[chunk_slice(0)], idx_v.at[0])
    pltpu.async_copy(table_hbm.at[idx_v.at[0]], rows_v.at[0], gsem)

    def pair(p, carry):
        for b in range(2):
            i = p * 2 + b
            nxt = (b + 1) % 2
            i_nxt = lax.rem(i + 1, _NCHUNK)
            pltpu.sync_copy(idx_hbm.at[pl.ds(base0 + i_nxt * _CHUNK, _CHUNK)],
                            idx_v.at[nxt])
            pltpu.make_async_copy(table_hbm.at[idx_v.at[b]], rows_v.at[b],
                                  gsem).wait()
            pltpu.async_copy(table_hbm.at[idx_v.at[nxt]], rows_v.at[nxt], gsem)
            pltpu.sync_copy(rows_v.at[b], out_hbm.at[chunk_slice(i)])
        return carry

    lax.fori_loop(0, _NCHUNK // 2, pair, 0)
    pltpu.make_async_copy(table_hbm.at[idx_v.at[0]], rows_v.at[0], gsem).wait()


@functools.cache
def _sc_gather():
    return functools.partial(
        pl.kernel,
        mesh=plsc.VectorSubcoreMesh(core_axis_name="c", subcore_axis_name="s"),
        compiler_params=pltpu.CompilerParams(use_tc_tiling_on_sc=False),
        out_type=jax.ShapeDtypeStruct((NROWS, D), jnp.float32),
        scratch_types=[
            pltpu.VMEM((2, _CHUNK), jnp.int32),
            pltpu.VMEM((2, _CHUNK, D), jnp.float32),
            pltpu.SemaphoreType.DMA,
        ],
    )(_gather_body)


def kernel(xs, table):
    idx = xs.reshape(NROWS).astype(jnp.int32)
    ntop = _noise_half(0)
    nbot = _noise_half(_LDH)
    emb = _sc_gather()(idx, table)
    out_t = _add_t(emb.reshape(NROWS // 2, 128), ntop, nbot)
    return out_t.reshape(L, D, B).transpose(2, 0, 1)


# speedup vs baseline: 2.0235x; 1.1393x over previous
import functools
import math

import jax
import jax.numpy as jnp
from jax import lax
from jax.experimental import pallas as pl
from jax.experimental.pallas import tpu as pltpu
from jax.experimental.pallas import tpu_sc as plsc

NUM_EMB = 1000000
D = 64
B = 4096
L = 200
NROWS = B * L
ALPHA = 5.0
SCALE = ALPHA / math.sqrt(L * D)

KS0 = 0
KS1 = 12345
KS2 = KS0 ^ KS1 ^ 0x1BD11BDA

_ROT = ((13, 15, 26, 6), (17, 29, 16, 24))


def _rotl(x, r):
    return lax.shift_left(x, jnp.int32(r)) | lax.shift_right_logical(
        x, jnp.int32(32 - r))


def _threefry_bits(cnt):
    ks = (jnp.int32(KS0), jnp.int32(KS1), jnp.int32(KS2))
    x0 = jnp.zeros_like(cnt) + ks[0]
    x1 = cnt + ks[1]
    for i in range(5):
        for r in _ROT[i % 2]:
            x0 = x0 + x1
            x1 = _rotl(x1, r)
            x1 = x1 ^ x0
        x0 = x0 + ks[(i + 1) % 3]
        x1 = x1 + ks[(i + 2) % 3] + jnp.int32(i + 1)
    return x0 ^ x1


_TCB = 128
_LD = L * D


_LDH = _LD // 2


def _noise_half_body(r0, out_ref):
    g = pl.program_id(0)
    r = jax.lax.broadcasted_iota(jnp.int32, (_LDH, _TCB), 0)
    c = jax.lax.broadcasted_iota(jnp.int32, (_LDH, _TCB), 1)
    cnt = (g * _TCB) * _LD + c * _LD + r + r0
    bits = _threefry_bits(cnt)
    u = lax.shift_right_logical(bits, jnp.int32(9)) | jnp.int32(0x3F800000)
    eps = lax.bitcast_convert_type(u, jnp.float32) - 1.0
    out_ref[...] = jnp.float32(SCALE) * eps


def _noise_half(r0):
    return pl.pallas_call(
        functools.partial(_noise_half_body, r0),
        grid=(B // _TCB,),
        out_specs=pl.BlockSpec((_LDH, _TCB), lambda i: (0, i)),
        out_shape=jax.ShapeDtypeStruct((_LDH, B), jnp.float32),
        cost_estimate=pl.CostEstimate(
            flops=int(130 * _LDH * B), transcendentals=0,
            bytes_accessed=int(4 * _LDH * B)),
    )()


_TCR = _TCB * _LD // 128


def _add_body(emb_ref, ntop_ref, nbot_ref, out_ref):
    x = emb_ref[...]
    x3 = x.reshape(_TCB, _LD // 128, 128)
    xt = jnp.transpose(x3, (1, 2, 0)).reshape(_LD, _TCB)
    out_ref[pl.ds(0, _LDH), :] = xt[:_LDH, :] + ntop_ref[...]
    out_ref[pl.ds(_LDH, _LDH), :] = xt[_LDH:, :] + nbot_ref[...]


def _add_t(emb128, ntop, nbot):
    return pl.pallas_call(
        _add_body,
        grid=(B // _TCB,),
        in_specs=[pl.BlockSpec((_TCR, 128), lambda i: (i, 0)),
                  pl.BlockSpec((_LDH, _TCB), lambda i: (0, i)),
                  pl.BlockSpec((_LDH, _TCB), lambda i: (0, i))],
        out_specs=pl.BlockSpec((_LD, _TCB), lambda i: (0, i)),
        out_shape=jax.ShapeDtypeStruct((_LD, B), jnp.float32),
        cost_estimate=pl.CostEstimate(
            flops=int(_LD * B), transcendentals=0,
            bytes_accessed=int(3 * 4 * _LD * B)),
    )(emb128, ntop, nbot)


_NW = 32
_ROWS_PER_W = NROWS // _NW
_CHUNK = 512
_NCHUNK = _ROWS_PER_W // _CHUNK


def _gather_body(idx_hbm, table_hbm, out_hbm, idx_v, rows_v, gsem):
    wid = lax.axis_index("s") * 2 + lax.axis_index("c")
    base0 = wid * _ROWS_PER_W

    def chunk_slice(i):
        return pl.ds(base0 + i * _CHUNK, _CHUNK)

    pltpu.sync_copy(idx_hbm.at[chunk_slice(0)], idx_v.at[0])
    pltpu.async_copy(table_hbm.at[idx_v.at[0]], rows_v.at[0], gsem)

    def pair(p, carry):
        for b in range(2):
            i = p * 2 + b
            nxt = (b + 1) % 2
            i_nxt = lax.rem(i + 1, _NCHUNK)
            pltpu.sync_copy(idx_hbm.at[pl.ds(base0 + i_nxt * _CHUNK, _CHUNK)],
                            idx_v.at[nxt])
            pltpu.make_async_copy(table_hbm.at[idx_v.at[b]], rows_v.at[b],
                                  gsem).wait()
            pltpu.async_copy(table_hbm.at[idx_v.at[nxt]], rows_v.at[nxt], gsem)
            pltpu.sync_copy(rows_v.at[b], out_hbm.at[chunk_slice(i)])
        return carry

    lax.fori_loop(0, _NCHUNK // 2, pair, 0)
    pltpu.make_async_copy(table_hbm.at[idx_v.at[0]], rows_v.at[0], gsem).wait()


@functools.cache
def _sc_gather():
    return functools.partial(
        pl.kernel,
        mesh=plsc.VectorSubcoreMesh(core_axis_name="c", subcore_axis_name="s"),
        compiler_params=pltpu.CompilerParams(use_tc_tiling_on_sc=False),
        out_type=jax.ShapeDtypeStruct((NROWS, D), jnp.float32),
        scratch_types=[
            pltpu.VMEM((2, _CHUNK), jnp.int32),
            pltpu.VMEM((2, _CHUNK, D), jnp.float32),
            pltpu.SemaphoreType.DMA,
        ],
    )(_gather_body)


def kernel(xs, table):
    idx = xs.reshape(NROWS).astype(jnp.int32)
    ntop = _noise_half(0)
    idx_seq = lax.optimization_barrier((idx, ntop))[0]
    nbot = _noise_half(_LDH)
    emb = _sc_gather()(idx_seq, table)
    out_t = _add_t(emb.reshape(NROWS // 2, 128), ntop, nbot)
    return out_t.reshape(L, D, B).transpose(2, 0, 1)
